# Initial kernel scaffold; baseline (speedup 1.0000x reference)
import jax
import jax.numpy as jnp
from jax.experimental import pallas as pl


def kernel(node_features, senders, receivers, edge_features, W_kernel, W_bias, We_kernel, We_bias):
    raise NotImplementedError("write your pallas kernel here")


# speedup vs baseline: 5.9073x; 5.9073x over previous
import jax
import jax.numpy as jnp
from jax import lax
from jax.experimental import pallas as pl
from jax.experimental.pallas import tpu as pltpu
from jax.experimental.pallas import tpu_sc as plsc

N_NODES = 10000
N_EDGES = 320000
D = 128
DE = 16
NC = 2
NS = 16
LANES = 16
EDGES_PER_TILE = N_EDGES // (NC * NS)
CH = 128
NFULL = EDGES_PER_TILE // CH
TAIL = EDGES_PER_TILE - NFULL * CH
ZCH = 624
ROW_TAIL = N_NODES - NS * ZCH

_mesh = plsc.VectorSubcoreMesh(core_axis_name="c", subcore_axis_name="s")


def _zero_vmem_rows(ref, nrows, ncols):
    def body(i, carry):
        for j in range(ncols // LANES):
            ref[i, pl.ds(j * LANES, LANES)] = jnp.zeros((LANES,), jnp.float32)
        return carry
    lax.fori_loop(0, nrows, body, None)


def _deg_body(recv_hbm, degp_hbm, deg_sh, ones_v, idx_v, idx_t, zeros_v):
    c = lax.axis_index("c")
    t = lax.axis_index("s")
    for j in range(CH // LANES):
        ones_v[pl.ds(j * LANES, LANES)] = jnp.ones((LANES,), jnp.float32)
    for j in range(ZCH // LANES):
        zeros_v[pl.ds(j * LANES, LANES)] = jnp.zeros((LANES,), jnp.float32)
    pltpu.sync_copy(zeros_v.at[pl.ds(0, ZCH)], deg_sh.at[pl.ds(t * ZCH, ZCH)])

    @pl.when(t == NS - 1)
    def _():
        pltpu.sync_copy(zeros_v.at[pl.ds(0, ROW_TAIL)],
                        deg_sh.at[pl.ds(NS * ZCH, ROW_TAIL)])

    plsc.subcore_barrier()
    base = (c * NS + t) * EDGES_PER_TILE

    def chunk(k, carry):
        pltpu.sync_copy(recv_hbm.at[pl.ds(base + k * CH, CH)], idx_v)
        pltpu.sync_copy(ones_v, deg_sh.at[idx_v], add=True)
        return carry

    lax.fori_loop(0, NFULL, chunk, None)
    pltpu.sync_copy(recv_hbm.at[pl.ds(base + NFULL * CH, TAIL)], idx_t)
    pltpu.sync_copy(ones_v.at[pl.ds(0, TAIL)], deg_sh.at[idx_t], add=True)

    plsc.subcore_barrier()
    pltpu.sync_copy(deg_sh.at[pl.ds(t * ZCH, ZCH)], zeros_v)
    pltpu.sync_copy(zeros_v, degp_hbm.at[pl.ds(c * N_NODES + t * ZCH, ZCH)])

    @pl.when(t == NS - 1)
    def _():
        pltpu.sync_copy(deg_sh.at[pl.ds(NS * ZCH, ROW_TAIL)],
                        zeros_v.at[pl.ds(0, ROW_TAIL)])
        pltpu.sync_copy(zeros_v.at[pl.ds(0, ROW_TAIL)],
                        degp_hbm.at[pl.ds(c * N_NODES + NS * ZCH, ROW_TAIL)])


_deg_call = pl.kernel(
    _deg_body,
    out_type=jax.ShapeDtypeStruct((NC * N_NODES,), jnp.float32),
    mesh=_mesh,
    scratch_types=[
        pltpu.VMEM_SHARED((N_NODES,), jnp.float32),
        pltpu.VMEM((CH,), jnp.float32),
        pltpu.VMEM((CH,), jnp.int32),
        pltpu.VMEM((TAIL,), jnp.int32),
        pltpu.VMEM((ZCH,), jnp.float32),
    ],
)


BLK = 1000


def _proj_body(x_ref, w_ref, b_ref, degt_ref, g_ref, rs_ref):
    deg = degt_ref[:, 0:1] + degt_ref[:, 1:2]
    rs = lax.rsqrt(jnp.maximum(deg, 1.0))
    h = jnp.dot(x_ref[...], w_ref[...], preferred_element_type=jnp.float32)
    g_ref[...] = (h + b_ref[...]) * rs
    rs_ref[...] = rs


_proj_call = pl.pallas_call(
    _proj_body,
    grid=(N_NODES // BLK,),
    in_specs=[
        pl.BlockSpec((BLK, D), lambda i: (i, 0)),
        pl.BlockSpec((D, D), lambda i: (0, 0)),
        pl.BlockSpec((1, D), lambda i: (0, 0)),
        pl.BlockSpec((BLK, NC), lambda i: (i, 0)),
    ],
    out_specs=[
        pl.BlockSpec((BLK, D), lambda i: (i, 0)),
        pl.BlockSpec((BLK, 1), lambda i: (i, 0)),
    ],
    out_shape=[
        jax.ShapeDtypeStruct((N_NODES, D), jnp.float32),
        jax.ShapeDtypeStruct((N_NODES, 1), jnp.float32),
    ],
)


def _edge_body(g_hbm, rs_hbm, send_hbm, recv_hbm, eft_hbm,
               accp_hbm, aggp_hbm, sp_hbm,
               acc_sh, s_sh, rs_sh, agg_fs,
               rss_v, idx_s_v, idx_r_v, idx_s_t, idx_r_t, eft_v, wef_v,
               rows_v, sb_v):
    c = lax.axis_index("c")
    t = lax.axis_index("s")

    _zero_vmem_rows(rows_v, CH, D)
    for j in range(ZCH // LANES):
        sb_v[pl.ds(j * LANES, LANES)] = jnp.zeros((LANES,), jnp.float32)
    for j in range(CH // LANES):
        rss_v[pl.ds(j * LANES, LANES)] = jnp.zeros((LANES,), jnp.float32)

    r0 = t * ZCH
    nfull = ZCH // CH
    rem = ZCH - nfull * CH
    spans = [(r0 + i * CH, CH) for i in range(nfull)] + [(r0 + nfull * CH, rem)]
    for start, cnt in spans:
        pltpu.sync_copy(rows_v.at[pl.ds(0, cnt)], acc_sh.at[pl.ds(start, cnt)])
    pltpu.sync_copy(sb_v, s_sh.at[pl.ds(r0, ZCH)])
    for f in range(DE):
        pltpu.sync_copy(sb_v, agg_fs[f].at[pl.ds(r0, ZCH)])

    @pl.when(t == NS - 1)
    def _():
        pltpu.sync_copy(rows_v.at[pl.ds(0, ROW_TAIL)],
                        acc_sh.at[pl.ds(NS * ZCH, ROW_TAIL)])
        pltpu.sync_copy(sb_v.at[pl.ds(0, ROW_TAIL)],
                        s_sh.at[pl.ds(NS * ZCH, ROW_TAIL)])
        for f in range(DE):
            pltpu.sync_copy(sb_v.at[pl.ds(0, ROW_TAIL)],
                            agg_fs[f].at[pl.ds(NS * ZCH, ROW_TAIL)])

    pltpu.sync_copy(rs_hbm.at[pl.ds(r0, ZCH)], sb_v)
    pltpu.sync_copy(sb_v, rs_sh.at[pl.ds(r0, ZCH)])

    @pl.when(t == NS - 1)
    def _():
        pltpu.sync_copy(rs_hbm.at[pl.ds(NS * ZCH, ROW_TAIL)],
                        sb_v.at[pl.ds(0, ROW_TAIL)])
        pltpu.sync_copy(sb_v.at[pl.ds(0, ROW_TAIL)],
                        rs_sh.at[pl.ds(NS * ZCH, ROW_TAIL)])

    plsc.subcore_barrier()

    base = (c * NS + t) * EDGES_PER_TILE

    def process(off, idx_s_ref, idx_r_ref, n):
        pltpu.sync_copy(send_hbm.at[pl.ds(off, n)], idx_s_ref)
        pltpu.sync_copy(recv_hbm.at[pl.ds(off, n)], idx_r_ref)
        for f in range(DE):
            pltpu.sync_copy(eft_hbm.at[pl.ds(f * N_EDGES + off, n)],
                            eft_v.at[pl.ds(f * CH, n)])
        if n == CH:
            pltpu.sync_copy(g_hbm.at[idx_s_ref], rows_v)
        else:
            pltpu.sync_copy(g_hbm.at[idx_s_ref], rows_v.at[pl.ds(0, n)])
        pltpu.sync_copy(rs_sh.at[idx_s_ref], rss_v.at[pl.ds(0, n)])

        def group(j, carry):
            rsvec = rss_v[pl.ds(j * LANES, LANES)]
            for f in range(DE):
                wef_v[pl.ds(f * CH + j * LANES, LANES)] = (
                    rsvec * eft_v[pl.ds(f * CH + j * LANES, LANES)])
            return carry

        lax.fori_loop(0, n // LANES, group, None)

        if n == CH:
            pltpu.sync_copy(rows_v, acc_sh.at[idx_r_ref], add=True)
            pltpu.sync_copy(rss_v, s_sh.at[idx_r_ref], add=True)
        else:
            pltpu.sync_copy(rows_v.at[pl.ds(0, n)], acc_sh.at[idx_r_ref],
                            add=True)
            pltpu.sync_copy(rss_v.at[pl.ds(0, n)], s_sh.at[idx_r_ref],
                            add=True)
        for f in range(DE):
            pltpu.sync_copy(wef_v.at[pl.ds(f * CH, n)],
                            agg_fs[f].at[idx_r_ref], add=True)

    def chunk(k, carry):
        process(base + k * CH, idx_s_v, idx_r_v, CH)
        return carry

    lax.fori_loop(0, NFULL, chunk, None)
    process(base + NFULL * CH, idx_s_t, idx_r_t, TAIL)

    plsc.subcore_barrier()
    for start, cnt in spans:
        pltpu.sync_copy(acc_sh.at[pl.ds(start, cnt)], rows_v.at[pl.ds(0, cnt)])
        pltpu.sync_copy(rows_v.at[pl.ds(0, cnt)],
                        accp_hbm.at[c, pl.ds(start, cnt)])
    pltpu.sync_copy(s_sh.at[pl.ds(r0, ZCH)], sb_v)
    pltpu.sync_copy(sb_v, sp_hbm.at[pl.ds(c * N_NODES + r0, ZCH)])
    for f in range(DE):
        pltpu.sync_copy(agg_fs[f].at[pl.ds(r0, ZCH)], sb_v)
        pltpu.sync_copy(
            sb_v, aggp_hbm.at[pl.ds((c * DE + f) * N_NODES + r0, ZCH)])

    @pl.when(t == NS - 1)
    def _():
        pltpu.sync_copy(acc_sh.at[pl.ds(NS * ZCH, ROW_TAIL)],
                        rows_v.at[pl.ds(0, ROW_TAIL)])
        pltpu.sync_copy(rows_v.at[pl.ds(0, ROW_TAIL)],
                        accp_hbm.at[c, pl.ds(NS * ZCH, ROW_TAIL)])
        pltpu.sync_copy(s_sh.at[pl.ds(NS * ZCH, ROW_TAIL)],
                        sb_v.at[pl.ds(0, ROW_TAIL)])
        pltpu.sync_copy(sb_v.at[pl.ds(0, ROW_TAIL)],
                        sp_hbm.at[pl.ds(c * N_NODES + NS * ZCH, ROW_TAIL)])
        for f in range(DE):
            pltpu.sync_copy(agg_fs[f].at[pl.ds(NS * ZCH, ROW_TAIL)],
                            sb_v.at[pl.ds(0, ROW_TAIL)])
            pltpu.sync_copy(
                sb_v.at[pl.ds(0, ROW_TAIL)],
                aggp_hbm.at[pl.ds((c * DE + f) * N_NODES + NS * ZCH,
                                  ROW_TAIL)])


_edge_call = pl.kernel(
    _edge_body,
    out_type=(
        jax.ShapeDtypeStruct((NC, N_NODES, D), jnp.float32),
        jax.ShapeDtypeStruct((NC * DE * N_NODES,), jnp.float32),
        jax.ShapeDtypeStruct((NC * N_NODES,), jnp.float32),
    ),
    mesh=_mesh,
    scratch_types=[
        pltpu.VMEM_SHARED((N_NODES, D), jnp.float32),
        pltpu.VMEM_SHARED((N_NODES,), jnp.float32),
        pltpu.VMEM_SHARED((N_NODES,), jnp.float32),
        tuple(pltpu.VMEM_SHARED((N_NODES,), jnp.float32) for _ in range(DE)),
        pltpu.VMEM((CH,), jnp.float32),
        pltpu.VMEM((CH,), jnp.int32),
        pltpu.VMEM((CH,), jnp.int32),
        pltpu.VMEM((TAIL,), jnp.int32),
        pltpu.VMEM((TAIL,), jnp.int32),
        pltpu.VMEM((DE * CH,), jnp.float32),
        pltpu.VMEM((DE * CH,), jnp.float32),
        pltpu.VMEM((CH, D), jnp.float32),
        pltpu.VMEM((ZCH,), jnp.float32),
    ],
)


def _final_body(accp_ref, aggx_ref, st_ref, rs_ref, we_ref, be_ref, out_ref):
    acc = accp_ref[0] + accp_ref[1]
    aggf = aggx_ref[:, 0, :] + aggx_ref[:, 1, :]
    s = st_ref[:, 0:1] + st_ref[:, 1:2]
    proj = jnp.dot(aggf, we_ref[...], preferred_element_type=jnp.float32)
    out_ref[...] = rs_ref[...] * (acc + proj + s * be_ref[...])


_final_call = pl.pallas_call(
    _final_body,
    grid=(N_NODES // BLK,),
    in_specs=[
        pl.BlockSpec((NC, BLK, D), lambda i: (0, i, 0)),
        pl.BlockSpec((BLK, NC, DE), lambda i: (i, 0, 0)),
        pl.BlockSpec((BLK, NC), lambda i: (i, 0)),
        pl.BlockSpec((BLK, 1), lambda i: (i, 0)),
        pl.BlockSpec((DE, D), lambda i: (0, 0)),
        pl.BlockSpec((1, D), lambda i: (0, 0)),
    ],
    out_specs=pl.BlockSpec((BLK, D), lambda i: (i, 0)),
    out_shape=jax.ShapeDtypeStruct((N_NODES, D), jnp.float32),
)


def kernel(node_features, senders, receivers, edge_features,
           W_kernel, W_bias, We_kernel, We_bias):
    degp = _deg_call(receivers)
    degt = degp.reshape(NC, N_NODES).T
    g, rs2 = _proj_call(node_features, W_kernel,
                        W_bias.reshape(1, D), degt)
    rs1 = rs2.reshape(N_NODES)
    eft = edge_features.T.reshape(DE * N_EDGES)
    accp, aggp, sp = _edge_call(g, rs1, senders, receivers, eft)
    aggx = aggp.reshape(NC, DE, N_NODES).transpose(2, 0, 1)
    st = sp.reshape(NC, N_NODES).T
    return _final_call(accp, aggx, st, rs2, We_kernel, We_bias.reshape(1, D))


# speedup vs baseline: 13.0307x; 2.2059x over previous
import jax
import jax.numpy as jnp
from jax import lax
from jax.experimental import pallas as pl
from jax.experimental.pallas import tpu as pltpu
from jax.experimental.pallas import tpu_sc as plsc

N_NODES = 10000
N_EDGES = 320000
D = 128
DE = 16
NC = 2
NS = 16
LANES = 16
EDGES_PER_TILE = N_EDGES // (NC * NS)
CH = 128
NFULL = EDGES_PER_TILE // CH
TAIL = EDGES_PER_TILE - NFULL * CH
ZCH = 624
ROW_TAIL = N_NODES - NS * ZCH

_mesh = plsc.VectorSubcoreMesh(core_axis_name="c", subcore_axis_name="s")


def _zero_vmem_rows(ref, nrows, ncols):
    def body(i, carry):
        for j in range(ncols // LANES):
            ref[i, pl.ds(j * LANES, LANES)] = jnp.zeros((LANES,), jnp.float32)
        return carry
    lax.fori_loop(0, nrows, body, None)


def _deg_body(recv_hbm, degp_hbm, deg_sh, ones_v, idx_v, idx_t, zeros_v):
    c = lax.axis_index("c")
    t = lax.axis_index("s")
    for j in range(CH // LANES):
        ones_v[pl.ds(j * LANES, LANES)] = jnp.ones((LANES,), jnp.float32)
    for j in range(ZCH // LANES):
        zeros_v[pl.ds(j * LANES, LANES)] = jnp.zeros((LANES,), jnp.float32)
    pltpu.sync_copy(zeros_v.at[pl.ds(0, ZCH)], deg_sh.at[pl.ds(t * ZCH, ZCH)])

    @pl.when(t == NS - 1)
    def _():
        pltpu.sync_copy(zeros_v.at[pl.ds(0, ROW_TAIL)],
                        deg_sh.at[pl.ds(NS * ZCH, ROW_TAIL)])

    plsc.subcore_barrier()
    base = (c * NS + t) * EDGES_PER_TILE

    def chunk(k, carry):
        pltpu.sync_copy(recv_hbm.at[pl.ds(base + k * CH, CH)], idx_v)
        pltpu.sync_copy(ones_v, deg_sh.at[idx_v], add=True)
        return carry

    lax.fori_loop(0, NFULL, chunk, None)
    pltpu.sync_copy(recv_hbm.at[pl.ds(base + NFULL * CH, TAIL)], idx_t)
    pltpu.sync_copy(ones_v.at[pl.ds(0, TAIL)], deg_sh.at[idx_t], add=True)

    plsc.subcore_barrier()
    pltpu.sync_copy(deg_sh.at[pl.ds(t * ZCH, ZCH)], zeros_v)
    pltpu.sync_copy(zeros_v, degp_hbm.at[pl.ds(c * N_NODES + t * ZCH, ZCH)])

    @pl.when(t == NS - 1)
    def _():
        pltpu.sync_copy(deg_sh.at[pl.ds(NS * ZCH, ROW_TAIL)],
                        zeros_v.at[pl.ds(0, ROW_TAIL)])
        pltpu.sync_copy(zeros_v.at[pl.ds(0, ROW_TAIL)],
                        degp_hbm.at[pl.ds(c * N_NODES + NS * ZCH, ROW_TAIL)])


_deg_call = pl.kernel(
    _deg_body,
    out_type=jax.ShapeDtypeStruct((NC * N_NODES,), jnp.float32),
    mesh=_mesh,
    scratch_types=[
        pltpu.VMEM_SHARED((N_NODES,), jnp.float32),
        pltpu.VMEM((CH,), jnp.float32),
        pltpu.VMEM((CH,), jnp.int32),
        pltpu.VMEM((TAIL,), jnp.int32),
        pltpu.VMEM((ZCH,), jnp.float32),
    ],
)


BLK = 1000


def _proj_body(x_ref, w_ref, b_ref, degt_ref, g_ref, rs_ref):
    deg = degt_ref[:, 0:1] + degt_ref[:, 1:2]
    rs = lax.rsqrt(jnp.maximum(deg, 1.0))
    h = jnp.dot(x_ref[...], w_ref[...], preferred_element_type=jnp.float32)
    g_ref[...] = (h + b_ref[...]) * rs
    rs_ref[...] = rs


_proj_call = pl.pallas_call(
    _proj_body,
    grid=(N_NODES // BLK,),
    in_specs=[
        pl.BlockSpec((BLK, D), lambda i: (i, 0)),
        pl.BlockSpec((D, D), lambda i: (0, 0)),
        pl.BlockSpec((1, D), lambda i: (0, 0)),
        pl.BlockSpec((BLK, NC), lambda i: (i, 0)),
    ],
    out_specs=[
        pl.BlockSpec((BLK, D), lambda i: (i, 0)),
        pl.BlockSpec((BLK, 1), lambda i: (i, 0)),
    ],
    out_shape=[
        jax.ShapeDtypeStruct((N_NODES, D), jnp.float32),
        jax.ShapeDtypeStruct((N_NODES, 1), jnp.float32),
    ],
)


def _edge_body(g_hbm, rs_hbm, send_hbm, recv_hbm, eft_hbm,
               accp_hbm, aggp_hbm, sp_hbm,
               acc_sh, s_sh, rs_sh, agg_fs,
               rss_v, idx_s_v, idx_r_v, idx_s_t, idx_r_t, eft_v, wef_v,
               rows_v, sb_v, sem_idx, sem_ef, sem_gat, sem_sc):
    c = lax.axis_index("c")
    t = lax.axis_index("s")

    _zero_vmem_rows(rows_v, CH, D)
    for j in range(ZCH // LANES):
        sb_v[pl.ds(j * LANES, LANES)] = jnp.zeros((LANES,), jnp.float32)
    for j in range(CH // LANES):
        rss_v[pl.ds(j * LANES, LANES)] = jnp.zeros((LANES,), jnp.float32)

    r0 = t * ZCH
    nfull = ZCH // CH
    rem = ZCH - nfull * CH
    spans = [(r0 + i * CH, CH) for i in range(nfull)] + [(r0 + nfull * CH, rem)]
    for start, cnt in spans:
        pltpu.sync_copy(rows_v.at[pl.ds(0, cnt)], acc_sh.at[pl.ds(start, cnt)])
    pltpu.sync_copy(sb_v, s_sh.at[pl.ds(r0, ZCH)])
    for f in range(DE):
        pltpu.sync_copy(sb_v, agg_fs[f].at[pl.ds(r0, ZCH)])

    @pl.when(t == NS - 1)
    def _():
        pltpu.sync_copy(rows_v.at[pl.ds(0, ROW_TAIL)],
                        acc_sh.at[pl.ds(NS * ZCH, ROW_TAIL)])
        pltpu.sync_copy(sb_v.at[pl.ds(0, ROW_TAIL)],
                        s_sh.at[pl.ds(NS * ZCH, ROW_TAIL)])
        for f in range(DE):
            pltpu.sync_copy(sb_v.at[pl.ds(0, ROW_TAIL)],
                            agg_fs[f].at[pl.ds(NS * ZCH, ROW_TAIL)])

    pltpu.sync_copy(rs_hbm.at[pl.ds(r0, ZCH)], sb_v)
    pltpu.sync_copy(sb_v, rs_sh.at[pl.ds(r0, ZCH)])

    @pl.when(t == NS - 1)
    def _():
        pltpu.sync_copy(rs_hbm.at[pl.ds(NS * ZCH, ROW_TAIL)],
                        sb_v.at[pl.ds(0, ROW_TAIL)])
        pltpu.sync_copy(sb_v.at[pl.ds(0, ROW_TAIL)],
                        rs_sh.at[pl.ds(NS * ZCH, ROW_TAIL)])

    plsc.subcore_barrier()

    base = (c * NS + t) * EDGES_PER_TILE

    def process(off, idx_s_ref, idx_r_ref, n):
        c_is = pltpu.async_copy(send_hbm.at[pl.ds(off, n)], idx_s_ref,
                                sem_idx)
        c_ir = pltpu.async_copy(recv_hbm.at[pl.ds(off, n)], idx_r_ref,
                                sem_idx)
        c_ef = [
            pltpu.async_copy(eft_hbm.at[pl.ds(f * N_EDGES + off, n)],
                             eft_v.at[pl.ds(f * CH, n)], sem_ef)
            for f in range(DE)
        ]
        c_is.wait()
        c_ir.wait()
        rows_dst = rows_v if n == CH else rows_v.at[pl.ds(0, n)]
        pltpu.sync_copy(g_hbm.at[idx_s_ref], rows_dst)
        pltpu.sync_copy(rs_sh.at[idx_s_ref], rss_v.at[pl.ds(0, n)])
        for cp in c_ef:
            cp.wait()

        def group(j, carry):
            rsvec = rss_v[pl.ds(j * LANES, LANES)]
            for f in range(DE):
                wef_v[pl.ds(f * CH + j * LANES, LANES)] = (
                    rsvec * eft_v[pl.ds(f * CH + j * LANES, LANES)])
            return carry

        lax.fori_loop(0, n // LANES, group, None)

        pltpu.sync_copy(rows_dst, acc_sh.at[idx_r_ref], add=True)
        pltpu.sync_copy(rss_v.at[pl.ds(0, n)], s_sh.at[idx_r_ref], add=True)
        for f in range(DE):
            pltpu.sync_copy(wef_v.at[pl.ds(f * CH, n)],
                            agg_fs[f].at[idx_r_ref], add=True)

    def chunk(k, carry):
        process(base + k * CH, idx_s_v, idx_r_v, CH)
        return carry

    lax.fori_loop(0, NFULL, chunk, None)
    process(base + NFULL * CH, idx_s_t, idx_r_t, TAIL)

    plsc.subcore_barrier()
    for start, cnt in spans:
        pltpu.sync_copy(acc_sh.at[pl.ds(start, cnt)], rows_v.at[pl.ds(0, cnt)])
        pltpu.sync_copy(rows_v.at[pl.ds(0, cnt)],
                        accp_hbm.at[c, pl.ds(start, cnt)])
    pltpu.sync_copy(s_sh.at[pl.ds(r0, ZCH)], sb_v)
    pltpu.sync_copy(sb_v, sp_hbm.at[pl.ds(c * N_NODES + r0, ZCH)])
    for f in range(DE):
        pltpu.sync_copy(agg_fs[f].at[pl.ds(r0, ZCH)], sb_v)
        pltpu.sync_copy(
            sb_v, aggp_hbm.at[pl.ds((c * DE + f) * N_NODES + r0, ZCH)])

    @pl.when(t == NS - 1)
    def _():
        pltpu.sync_copy(acc_sh.at[pl.ds(NS * ZCH, ROW_TAIL)],
                        rows_v.at[pl.ds(0, ROW_TAIL)])
        pltpu.sync_copy(rows_v.at[pl.ds(0, ROW_TAIL)],
                        accp_hbm.at[c, pl.ds(NS * ZCH, ROW_TAIL)])
        pltpu.sync_copy(s_sh.at[pl.ds(NS * ZCH, ROW_TAIL)],
                        sb_v.at[pl.ds(0, ROW_TAIL)])
        pltpu.sync_copy(sb_v.at[pl.ds(0, ROW_TAIL)],
                        sp_hbm.at[pl.ds(c * N_NODES + NS * ZCH, ROW_TAIL)])
        for f in range(DE):
            pltpu.sync_copy(agg_fs[f].at[pl.ds(NS * ZCH, ROW_TAIL)],
                            sb_v.at[pl.ds(0, ROW_TAIL)])
            pltpu.sync_copy(
                sb_v.at[pl.ds(0, ROW_TAIL)],
                aggp_hbm.at[pl.ds((c * DE + f) * N_NODES + NS * ZCH,
                                  ROW_TAIL)])


_edge_call = pl.kernel(
    _edge_body,
    out_type=(
        jax.ShapeDtypeStruct((NC, N_NODES, D), jnp.float32),
        jax.ShapeDtypeStruct((NC * DE * N_NODES,), jnp.float32),
        jax.ShapeDtypeStruct((NC * N_NODES,), jnp.float32),
    ),
    mesh=_mesh,
    scratch_types=[
        pltpu.VMEM_SHARED((N_NODES, D), jnp.float32),
        pltpu.VMEM_SHARED((N_NODES,), jnp.float32),
        pltpu.VMEM_SHARED((N_NODES,), jnp.float32),
        tuple(pltpu.VMEM_SHARED((N_NODES,), jnp.float32) for _ in range(DE)),
        pltpu.VMEM((CH,), jnp.float32),
        pltpu.VMEM((CH,), jnp.int32),
        pltpu.VMEM((CH,), jnp.int32),
        pltpu.VMEM((TAIL,), jnp.int32),
        pltpu.VMEM((TAIL,), jnp.int32),
        pltpu.VMEM((DE * CH,), jnp.float32),
        pltpu.VMEM((DE * CH,), jnp.float32),
        pltpu.VMEM((CH, D), jnp.float32),
        pltpu.VMEM((ZCH,), jnp.float32),
        pltpu.SemaphoreType.DMA,
        pltpu.SemaphoreType.DMA,
        pltpu.SemaphoreType.DMA,
        pltpu.SemaphoreType.DMA,
    ],
)


def _final_body(accp_ref, aggx_ref, st_ref, rs_ref, we_ref, be_ref, out_ref):
    acc = accp_ref[0] + accp_ref[1]
    aggf = aggx_ref[:, 0, :] + aggx_ref[:, 1, :]
    s = st_ref[:, 0:1] + st_ref[:, 1:2]
    proj = jnp.dot(aggf, we_ref[...], preferred_element_type=jnp.float32)
    out_ref[...] = rs_ref[...] * (acc + proj + s * be_ref[...])


_final_call = pl.pallas_call(
    _final_body,
    grid=(N_NODES // BLK,),
    in_specs=[
        pl.BlockSpec((NC, BLK, D), lambda i: (0, i, 0)),
        pl.BlockSpec((BLK, NC, DE), lambda i: (i, 0, 0)),
        pl.BlockSpec((BLK, NC), lambda i: (i, 0)),
        pl.BlockSpec((BLK, 1), lambda i: (i, 0)),
        pl.BlockSpec((DE, D), lambda i: (0, 0)),
        pl.BlockSpec((1, D), lambda i: (0, 0)),
    ],
    out_specs=pl.BlockSpec((BLK, D), lambda i: (i, 0)),
    out_shape=jax.ShapeDtypeStruct((N_NODES, D), jnp.float32),
)


def kernel(node_features, senders, receivers, edge_features,
           W_kernel, W_bias, We_kernel, We_bias):
    degp = _deg_call(receivers)
    degt = degp.reshape(NC, N_NODES).T
    g, rs2 = _proj_call(node_features, W_kernel,
                        W_bias.reshape(1, D), degt)
    rs1 = rs2.reshape(N_NODES)
    eft = edge_features.T.reshape(DE * N_EDGES)
    accp, aggp, sp = _edge_call(g, rs1, senders, receivers, eft)
    aggx = aggp.reshape(NC, DE, N_NODES).transpose(2, 0, 1)
    st = sp.reshape(NC, N_NODES).T
    return _final_call(accp, aggx, st, rs2, We_kernel, We_bias.reshape(1, D))


# speedup vs baseline: 13.7392x; 1.0544x over previous
import jax
import jax.numpy as jnp
from jax import lax
from jax.experimental import pallas as pl
from jax.experimental.pallas import tpu as pltpu
from jax.experimental.pallas import tpu_sc as plsc

N_NODES = 10000
N_EDGES = 320000
D = 128
DE = 16
NC = 2
NS = 16
LANES = 16
EDGES_PER_TILE = N_EDGES // (NC * NS)
CH = 128
NFULL = EDGES_PER_TILE // CH
TAIL = EDGES_PER_TILE - NFULL * CH
ZCH = 624
ROW_TAIL = N_NODES - NS * ZCH

_mesh = plsc.VectorSubcoreMesh(core_axis_name="c", subcore_axis_name="s")


def _zero_vmem_rows(ref, nrows, ncols):
    def body(i, carry):
        for j in range(ncols // LANES):
            ref[i, pl.ds(j * LANES, LANES)] = jnp.zeros((LANES,), jnp.float32)
        return carry
    lax.fori_loop(0, nrows, body, None)


def _deg_body(recv_hbm, degp_hbm, deg_sh, ones_v, idx_v, idx_t, zeros_v):
    c = lax.axis_index("c")
    t = lax.axis_index("s")
    for j in range(CH // LANES):
        ones_v[pl.ds(j * LANES, LANES)] = jnp.ones((LANES,), jnp.float32)
    for j in range(ZCH // LANES):
        zeros_v[pl.ds(j * LANES, LANES)] = jnp.zeros((LANES,), jnp.float32)
    pltpu.sync_copy(zeros_v.at[pl.ds(0, ZCH)], deg_sh.at[pl.ds(t * ZCH, ZCH)])

    @pl.when(t == NS - 1)
    def _():
        pltpu.sync_copy(zeros_v.at[pl.ds(0, ROW_TAIL)],
                        deg_sh.at[pl.ds(NS * ZCH, ROW_TAIL)])

    plsc.subcore_barrier()
    base = (c * NS + t) * EDGES_PER_TILE

    def chunk(k, carry):
        pltpu.sync_copy(recv_hbm.at[pl.ds(base + k * CH, CH)], idx_v)
        pltpu.sync_copy(ones_v, deg_sh.at[idx_v], add=True)
        return carry

    lax.fori_loop(0, NFULL, chunk, None)
    pltpu.sync_copy(recv_hbm.at[pl.ds(base + NFULL * CH, TAIL)], idx_t)
    pltpu.sync_copy(ones_v.at[pl.ds(0, TAIL)], deg_sh.at[idx_t], add=True)

    plsc.subcore_barrier()
    pltpu.sync_copy(deg_sh.at[pl.ds(t * ZCH, ZCH)], zeros_v)
    pltpu.sync_copy(zeros_v, degp_hbm.at[pl.ds(c * N_NODES + t * ZCH, ZCH)])

    @pl.when(t == NS - 1)
    def _():
        pltpu.sync_copy(deg_sh.at[pl.ds(NS * ZCH, ROW_TAIL)],
                        zeros_v.at[pl.ds(0, ROW_TAIL)])
        pltpu.sync_copy(zeros_v.at[pl.ds(0, ROW_TAIL)],
                        degp_hbm.at[pl.ds(c * N_NODES + NS * ZCH, ROW_TAIL)])


_deg_call = pl.kernel(
    _deg_body,
    out_type=jax.ShapeDtypeStruct((NC * N_NODES,), jnp.float32),
    mesh=_mesh,
    scratch_types=[
        pltpu.VMEM_SHARED((N_NODES,), jnp.float32),
        pltpu.VMEM((CH,), jnp.float32),
        pltpu.VMEM((CH,), jnp.int32),
        pltpu.VMEM((TAIL,), jnp.int32),
        pltpu.VMEM((ZCH,), jnp.float32),
    ],
)


BLK = 1000


def _proj_body(x_ref, w_ref, b_ref, degt_ref, g_ref, rs_ref):
    deg = degt_ref[:, 0:1] + degt_ref[:, 1:2]
    rs = lax.rsqrt(jnp.maximum(deg, 1.0))
    h = jnp.dot(x_ref[...], w_ref[...], preferred_element_type=jnp.float32)
    g_ref[...] = (h + b_ref[...]) * rs
    rs_ref[...] = rs


_proj_call = pl.pallas_call(
    _proj_body,
    grid=(N_NODES // BLK,),
    in_specs=[
        pl.BlockSpec((BLK, D), lambda i: (i, 0)),
        pl.BlockSpec((D, D), lambda i: (0, 0)),
        pl.BlockSpec((1, D), lambda i: (0, 0)),
        pl.BlockSpec((BLK, NC), lambda i: (i, 0)),
    ],
    out_specs=[
        pl.BlockSpec((BLK, D), lambda i: (i, 0)),
        pl.BlockSpec((BLK, 1), lambda i: (i, 0)),
    ],
    out_shape=[
        jax.ShapeDtypeStruct((N_NODES, D), jnp.float32),
        jax.ShapeDtypeStruct((N_NODES, 1), jnp.float32),
    ],
)


def _edge_body(g_hbm, rs_hbm, send_hbm, recv_hbm, eft_hbm,
               accp_hbm, aggp_hbm, sp_hbm,
               acc_sh, s_sh, rs_sh, agg_fs,
               rss_v, idx_s2, idx_r2, idx_s_t, idx_r_t, eft2, wef_v,
               rows_v, sb_v, sem_pre2):
    c = lax.axis_index("c")
    t = lax.axis_index("s")

    _zero_vmem_rows(rows_v, CH, D)
    for j in range(ZCH // LANES):
        sb_v[pl.ds(j * LANES, LANES)] = jnp.zeros((LANES,), jnp.float32)
    for j in range(CH // LANES):
        rss_v[pl.ds(j * LANES, LANES)] = jnp.zeros((LANES,), jnp.float32)

    r0 = t * ZCH
    nfull = ZCH // CH
    rem = ZCH - nfull * CH
    spans = [(r0 + i * CH, CH) for i in range(nfull)] + [(r0 + nfull * CH, rem)]
    for start, cnt in spans:
        pltpu.sync_copy(rows_v.at[pl.ds(0, cnt)], acc_sh.at[pl.ds(start, cnt)])
    pltpu.sync_copy(sb_v, s_sh.at[pl.ds(r0, ZCH)])
    for f in range(DE):
        pltpu.sync_copy(sb_v, agg_fs[f].at[pl.ds(r0, ZCH)])

    @pl.when(t == NS - 1)
    def _():
        pltpu.sync_copy(rows_v.at[pl.ds(0, ROW_TAIL)],
                        acc_sh.at[pl.ds(NS * ZCH, ROW_TAIL)])
        pltpu.sync_copy(sb_v.at[pl.ds(0, ROW_TAIL)],
                        s_sh.at[pl.ds(NS * ZCH, ROW_TAIL)])
        for f in range(DE):
            pltpu.sync_copy(sb_v.at[pl.ds(0, ROW_TAIL)],
                            agg_fs[f].at[pl.ds(NS * ZCH, ROW_TAIL)])

    pltpu.sync_copy(rs_hbm.at[pl.ds(r0, ZCH)], sb_v)
    pltpu.sync_copy(sb_v, rs_sh.at[pl.ds(r0, ZCH)])

    @pl.when(t == NS - 1)
    def _():
        pltpu.sync_copy(rs_hbm.at[pl.ds(NS * ZCH, ROW_TAIL)],
                        sb_v.at[pl.ds(0, ROW_TAIL)])
        pltpu.sync_copy(sb_v.at[pl.ds(0, ROW_TAIL)],
                        rs_sh.at[pl.ds(NS * ZCH, ROW_TAIL)])

    plsc.subcore_barrier()

    base = (c * NS + t) * EDGES_PER_TILE

    def _pre_copies(kk, p, make):
        off = base + kk * CH
        pairs = [(send_hbm.at[pl.ds(off, CH)], idx_s2[p]),
                 (recv_hbm.at[pl.ds(off, CH)], idx_r2[p])]
        pairs += [(eft_hbm.at[pl.ds(f * N_EDGES + off, CH)],
                   eft2[p].at[pl.ds(f * CH, CH)]) for f in range(DE)]
        for s_ref, d_ref in pairs:
            if make:
                pltpu.make_async_copy(s_ref, d_ref, sem_pre2[p]).wait()
            else:
                pltpu.async_copy(s_ref, d_ref, sem_pre2[p])

    def chunk_body(kk, p, prefetch_next):
        _pre_copies(kk, p, True)
        pltpu.sync_copy(g_hbm.at[idx_s2[p]], rows_v)

        @pl.when(prefetch_next)
        def _():
            _pre_copies(kk + 1, 1 - p, False)

        pltpu.sync_copy(rs_sh.at[idx_s2[p]], rss_v)

        def group(j, carry):
            rsvec = rss_v[pl.ds(j * LANES, LANES)]
            for f in range(DE):
                wef_v[pl.ds(f * CH + j * LANES, LANES)] = (
                    rsvec * eft2[p][pl.ds(f * CH + j * LANES, LANES)])
            return carry

        lax.fori_loop(0, CH // LANES, group, None)

        pltpu.sync_copy(rows_v, acc_sh.at[idx_r2[p]], add=True)
        pltpu.sync_copy(rss_v, s_sh.at[idx_r2[p]], add=True)
        for f in range(DE):
            pltpu.sync_copy(wef_v.at[pl.ds(f * CH, CH)],
                            agg_fs[f].at[idx_r2[p]], add=True)

    _pre_copies(0, 0, False)

    def pair(j0, carry):
        k = 2 * j0
        chunk_body(k, 0, True)
        chunk_body(k + 1, 1, j0 < NFULL // 2 - 1)
        return carry

    lax.fori_loop(0, NFULL // 2, pair, None)

    off_t = base + NFULL * CH
    pltpu.sync_copy(send_hbm.at[pl.ds(off_t, TAIL)], idx_s_t)
    pltpu.sync_copy(recv_hbm.at[pl.ds(off_t, TAIL)], idx_r_t)
    for f in range(DE):
        pltpu.sync_copy(eft_hbm.at[pl.ds(f * N_EDGES + off_t, TAIL)],
                        eft2[0].at[pl.ds(f * CH, TAIL)])
    pltpu.sync_copy(g_hbm.at[idx_s_t], rows_v.at[pl.ds(0, TAIL)])
    pltpu.sync_copy(rs_sh.at[idx_s_t], rss_v.at[pl.ds(0, TAIL)])
    rsvec_t = rss_v[pl.ds(0, LANES)]
    for f in range(DE):
        wef_v[pl.ds(f * CH, LANES)] = (
            rsvec_t * eft2[0][pl.ds(f * CH, LANES)])
    pltpu.sync_copy(rows_v.at[pl.ds(0, TAIL)], acc_sh.at[idx_r_t], add=True)
    pltpu.sync_copy(rss_v.at[pl.ds(0, TAIL)], s_sh.at[idx_r_t], add=True)
    for f in range(DE):
        pltpu.sync_copy(wef_v.at[pl.ds(f * CH, TAIL)], agg_fs[f].at[idx_r_t],
                        add=True)

    plsc.subcore_barrier()
    for start, cnt in spans:
        pltpu.sync_copy(acc_sh.at[pl.ds(start, cnt)], rows_v.at[pl.ds(0, cnt)])
        pltpu.sync_copy(rows_v.at[pl.ds(0, cnt)],
                        accp_hbm.at[c, pl.ds(start, cnt)])
    pltpu.sync_copy(s_sh.at[pl.ds(r0, ZCH)], sb_v)
    pltpu.sync_copy(sb_v, sp_hbm.at[pl.ds(c * N_NODES + r0, ZCH)])
    for f in range(DE):
        pltpu.sync_copy(agg_fs[f].at[pl.ds(r0, ZCH)], sb_v)
        pltpu.sync_copy(
            sb_v, aggp_hbm.at[pl.ds((c * DE + f) * N_NODES + r0, ZCH)])

    @pl.when(t == NS - 1)
    def _():
        pltpu.sync_copy(acc_sh.at[pl.ds(NS * ZCH, ROW_TAIL)],
                        rows_v.at[pl.ds(0, ROW_TAIL)])
        pltpu.sync_copy(rows_v.at[pl.ds(0, ROW_TAIL)],
                        accp_hbm.at[c, pl.ds(NS * ZCH, ROW_TAIL)])
        pltpu.sync_copy(s_sh.at[pl.ds(NS * ZCH, ROW_TAIL)],
                        sb_v.at[pl.ds(0, ROW_TAIL)])
        pltpu.sync_copy(sb_v.at[pl.ds(0, ROW_TAIL)],
                        sp_hbm.at[pl.ds(c * N_NODES + NS * ZCH, ROW_TAIL)])
        for f in range(DE):
            pltpu.sync_copy(agg_fs[f].at[pl.ds(NS * ZCH, ROW_TAIL)],
                            sb_v.at[pl.ds(0, ROW_TAIL)])
            pltpu.sync_copy(
                sb_v.at[pl.ds(0, ROW_TAIL)],
                aggp_hbm.at[pl.ds((c * DE + f) * N_NODES + NS * ZCH,
                                  ROW_TAIL)])


_edge_call = pl.kernel(
    _edge_body,
    out_type=(
        jax.ShapeDtypeStruct((NC, N_NODES, D), jnp.float32),
        jax.ShapeDtypeStruct((NC * DE * N_NODES,), jnp.float32),
        jax.ShapeDtypeStruct((NC * N_NODES,), jnp.float32),
    ),
    mesh=_mesh,
    scratch_types=[
        pltpu.VMEM_SHARED((N_NODES, D), jnp.float32),
        pltpu.VMEM_SHARED((N_NODES,), jnp.float32),
        pltpu.VMEM_SHARED((N_NODES,), jnp.float32),
        tuple(pltpu.VMEM_SHARED((N_NODES,), jnp.float32) for _ in range(DE)),
        pltpu.VMEM((CH,), jnp.float32),
        tuple(pltpu.VMEM((CH,), jnp.int32) for _ in range(2)),
        tuple(pltpu.VMEM((CH,), jnp.int32) for _ in range(2)),
        pltpu.VMEM((TAIL,), jnp.int32),
        pltpu.VMEM((TAIL,), jnp.int32),
        tuple(pltpu.VMEM((DE * CH,), jnp.float32) for _ in range(2)),
        pltpu.VMEM((DE * CH,), jnp.float32),
        pltpu.VMEM((CH, D), jnp.float32),
        pltpu.VMEM((ZCH,), jnp.float32),
        (pltpu.SemaphoreType.DMA, pltpu.SemaphoreType.DMA),
    ],
)


def _final_body(accp_ref, aggx_ref, st_ref, rs_ref, we_ref, be_ref, out_ref):
    acc = accp_ref[0] + accp_ref[1]
    aggf = aggx_ref[:, 0, :] + aggx_ref[:, 1, :]
    s = st_ref[:, 0:1] + st_ref[:, 1:2]
    proj = jnp.dot(aggf, we_ref[...], preferred_element_type=jnp.float32)
    out_ref[...] = rs_ref[...] * (acc + proj + s * be_ref[...])


_final_call = pl.pallas_call(
    _final_body,
    grid=(N_NODES // BLK,),
    in_specs=[
        pl.BlockSpec((NC, BLK, D), lambda i: (0, i, 0)),
        pl.BlockSpec((BLK, NC, DE), lambda i: (i, 0, 0)),
        pl.BlockSpec((BLK, NC), lambda i: (i, 0)),
        pl.BlockSpec((BLK, 1), lambda i: (i, 0)),
        pl.BlockSpec((DE, D), lambda i: (0, 0)),
        pl.BlockSpec((1, D), lambda i: (0, 0)),
    ],
    out_specs=pl.BlockSpec((BLK, D), lambda i: (i, 0)),
    out_shape=jax.ShapeDtypeStruct((N_NODES, D), jnp.float32),
)


def kernel(node_features, senders, receivers, edge_features,
           W_kernel, W_bias, We_kernel, We_bias):
    degp = _deg_call(receivers)
    degt = degp.reshape(NC, N_NODES).T
    g, rs2 = _proj_call(node_features, W_kernel,
                        W_bias.reshape(1, D), degt)
    rs1 = rs2.reshape(N_NODES)
    eft = edge_features.T.reshape(DE * N_EDGES)
    accp, aggp, sp = _edge_call(g, rs1, senders, receivers, eft)
    aggx = aggp.reshape(NC, DE, N_NODES).transpose(2, 0, 1)
    st = sp.reshape(NC, N_NODES).T
    return _final_call(accp, aggx, st, rs2, We_kernel, We_bias.reshape(1, D))


# speedup vs baseline: 13.9562x; 1.0158x over previous
import jax
import jax.numpy as jnp
from jax import lax
from jax.experimental import pallas as pl
from jax.experimental.pallas import tpu as pltpu
from jax.experimental.pallas import tpu_sc as plsc

N_NODES = 10000
N_EDGES = 320000
D = 128
DE = 16
NC = 2
NS = 16
LANES = 16
EDGES_PER_TILE = N_EDGES // (NC * NS)
CH = 128
NFULL = EDGES_PER_TILE // CH
TAIL = EDGES_PER_TILE - NFULL * CH
ZCH = 624
ROW_TAIL = N_NODES - NS * ZCH

_mesh = plsc.VectorSubcoreMesh(core_axis_name="c", subcore_axis_name="s")


def _zero_vmem_rows(ref, nrows, ncols):
    def body(i, carry):
        for j in range(ncols // LANES):
            ref[i, pl.ds(j * LANES, LANES)] = jnp.zeros((LANES,), jnp.float32)
        return carry
    lax.fori_loop(0, nrows, body, None)


def _deg_body(recv_hbm, degp_hbm, deg_sh, ones_v, idx2, idx_t, zeros_v,
              sem_d2):
    c = lax.axis_index("c")
    t = lax.axis_index("s")
    for j in range(CH // LANES):
        ones_v[pl.ds(j * LANES, LANES)] = jnp.ones((LANES,), jnp.float32)
    for j in range(ZCH // LANES):
        zeros_v[pl.ds(j * LANES, LANES)] = jnp.zeros((LANES,), jnp.float32)
    pltpu.sync_copy(zeros_v.at[pl.ds(0, ZCH)], deg_sh.at[pl.ds(t * ZCH, ZCH)])

    @pl.when(t == NS - 1)
    def _():
        pltpu.sync_copy(zeros_v.at[pl.ds(0, ROW_TAIL)],
                        deg_sh.at[pl.ds(NS * ZCH, ROW_TAIL)])

    plsc.subcore_barrier()
    base = (c * NS + t) * EDGES_PER_TILE

    def _idx_copy(kk, p, make):
        s_ref = recv_hbm.at[pl.ds(base + kk * CH, CH)]
        if make:
            pltpu.make_async_copy(s_ref, idx2[p], sem_d2[p]).wait()
        else:
            pltpu.async_copy(s_ref, idx2[p], sem_d2[p])

    def dchunk(kk, p, prefetch_next):
        _idx_copy(kk, p, True)

        @pl.when(prefetch_next)
        def _():
            _idx_copy(kk + 1, 1 - p, False)

        pltpu.sync_copy(ones_v, deg_sh.at[idx2[p]], add=True)

    _idx_copy(0, 0, False)

    def dpair(j0, carry):
        k = 2 * j0
        dchunk(k, 0, True)
        dchunk(k + 1, 1, j0 < NFULL // 2 - 1)
        return carry

    lax.fori_loop(0, NFULL // 2, dpair, None)
    pltpu.sync_copy(recv_hbm.at[pl.ds(base + NFULL * CH, TAIL)], idx_t)
    pltpu.sync_copy(ones_v.at[pl.ds(0, TAIL)], deg_sh.at[idx_t], add=True)

    plsc.subcore_barrier()
    pltpu.sync_copy(deg_sh.at[pl.ds(t * ZCH, ZCH)], zeros_v)
    pltpu.sync_copy(zeros_v, degp_hbm.at[pl.ds(c * N_NODES + t * ZCH, ZCH)])

    @pl.when(t == NS - 1)
    def _():
        pltpu.sync_copy(deg_sh.at[pl.ds(NS * ZCH, ROW_TAIL)],
                        zeros_v.at[pl.ds(0, ROW_TAIL)])
        pltpu.sync_copy(zeros_v.at[pl.ds(0, ROW_TAIL)],
                        degp_hbm.at[pl.ds(c * N_NODES + NS * ZCH, ROW_TAIL)])


_deg_call = pl.kernel(
    _deg_body,
    out_type=jax.ShapeDtypeStruct((NC * N_NODES,), jnp.float32),
    mesh=_mesh,
    scratch_types=[
        pltpu.VMEM_SHARED((N_NODES,), jnp.float32),
        pltpu.VMEM((CH,), jnp.float32),
        tuple(pltpu.VMEM((CH,), jnp.int32) for _ in range(2)),
        pltpu.VMEM((TAIL,), jnp.int32),
        pltpu.VMEM((ZCH,), jnp.float32),
        (pltpu.SemaphoreType.DMA, pltpu.SemaphoreType.DMA),
    ],
)


BLK = 1000


def _proj_body(x_ref, w_ref, b_ref, degt_ref, g_ref, rs_ref):
    deg = degt_ref[:, 0:1] + degt_ref[:, 1:2]
    rs = lax.rsqrt(jnp.maximum(deg, 1.0))
    h = jnp.dot(x_ref[...], w_ref[...], preferred_element_type=jnp.float32)
    g_ref[...] = (h + b_ref[...]) * rs
    rs_ref[...] = rs


_proj_call = pl.pallas_call(
    _proj_body,
    grid=(N_NODES // BLK,),
    in_specs=[
        pl.BlockSpec((BLK, D), lambda i: (i, 0)),
        pl.BlockSpec((D, D), lambda i: (0, 0)),
        pl.BlockSpec((1, D), lambda i: (0, 0)),
        pl.BlockSpec((BLK, NC), lambda i: (i, 0)),
    ],
    out_specs=[
        pl.BlockSpec((BLK, D), lambda i: (i, 0)),
        pl.BlockSpec((BLK, 1), lambda i: (i, 0)),
    ],
    out_shape=[
        jax.ShapeDtypeStruct((N_NODES, D), jnp.float32),
        jax.ShapeDtypeStruct((N_NODES, 1), jnp.float32),
    ],
)


def _edge_body(g_hbm, rs_hbm, send_hbm, recv_hbm, eft_hbm,
               accp_hbm, aggp_hbm, sp_hbm,
               acc_sh, s_sh, rs_sh, agg_fs,
               rss_v, idx_s2, idx_r2, idx_s_t, idx_r_t, eft2, wef_v,
               rows_v, sb_v, sem_pre2):
    c = lax.axis_index("c")
    t = lax.axis_index("s")

    _zero_vmem_rows(rows_v, CH, D)
    for j in range(ZCH // LANES):
        sb_v[pl.ds(j * LANES, LANES)] = jnp.zeros((LANES,), jnp.float32)
    for j in range(CH // LANES):
        rss_v[pl.ds(j * LANES, LANES)] = jnp.zeros((LANES,), jnp.float32)

    r0 = t * ZCH
    nfull = ZCH // CH
    rem = ZCH - nfull * CH
    spans = [(r0 + i * CH, CH) for i in range(nfull)] + [(r0 + nfull * CH, rem)]
    for start, cnt in spans:
        pltpu.sync_copy(rows_v.at[pl.ds(0, cnt)], acc_sh.at[pl.ds(start, cnt)])
    pltpu.sync_copy(sb_v, s_sh.at[pl.ds(r0, ZCH)])
    for f in range(DE):
        pltpu.sync_copy(sb_v, agg_fs[f].at[pl.ds(r0, ZCH)])

    @pl.when(t == NS - 1)
    def _():
        pltpu.sync_copy(rows_v.at[pl.ds(0, ROW_TAIL)],
                        acc_sh.at[pl.ds(NS * ZCH, ROW_TAIL)])
        pltpu.sync_copy(sb_v.at[pl.ds(0, ROW_TAIL)],
                        s_sh.at[pl.ds(NS * ZCH, ROW_TAIL)])
        for f in range(DE):
            pltpu.sync_copy(sb_v.at[pl.ds(0, ROW_TAIL)],
                            agg_fs[f].at[pl.ds(NS * ZCH, ROW_TAIL)])

    pltpu.sync_copy(rs_hbm.at[pl.ds(r0, ZCH)], sb_v)
    pltpu.sync_copy(sb_v, rs_sh.at[pl.ds(r0, ZCH)])

    @pl.when(t == NS - 1)
    def _():
        pltpu.sync_copy(rs_hbm.at[pl.ds(NS * ZCH, ROW_TAIL)],
                        sb_v.at[pl.ds(0, ROW_TAIL)])
        pltpu.sync_copy(sb_v.at[pl.ds(0, ROW_TAIL)],
                        rs_sh.at[pl.ds(NS * ZCH, ROW_TAIL)])

    plsc.subcore_barrier()

    base = (c * NS + t) * EDGES_PER_TILE

    def _pre_copies(kk, p, make):
        off = base + kk * CH
        pairs = [(send_hbm.at[pl.ds(off, CH)], idx_s2[p]),
                 (recv_hbm.at[pl.ds(off, CH)], idx_r2[p])]
        pairs += [(eft_hbm.at[pl.ds(f * N_EDGES + off, CH)],
                   eft2[p].at[pl.ds(f * CH, CH)]) for f in range(DE)]
        for s_ref, d_ref in pairs:
            if make:
                pltpu.make_async_copy(s_ref, d_ref, sem_pre2[p]).wait()
            else:
                pltpu.async_copy(s_ref, d_ref, sem_pre2[p])

    def chunk_body(kk, p, prefetch_next):
        _pre_copies(kk, p, True)
        pltpu.sync_copy(g_hbm.at[idx_s2[p]], rows_v)

        @pl.when(prefetch_next)
        def _():
            _pre_copies(kk + 1, 1 - p, False)

        pltpu.sync_copy(rs_sh.at[idx_s2[p]], rss_v)

        def group(j, carry):
            rsvec = rss_v[pl.ds(j * LANES, LANES)]
            for f in range(DE):
                wef_v[pl.ds(f * CH + j * LANES, LANES)] = (
                    rsvec * eft2[p][pl.ds(f * CH + j * LANES, LANES)])
            return carry

        lax.fori_loop(0, CH // LANES, group, None)

        pltpu.sync_copy(rows_v, acc_sh.at[idx_r2[p]], add=True)
        pltpu.sync_copy(rss_v, s_sh.at[idx_r2[p]], add=True)
        for f in range(DE):
            pltpu.sync_copy(wef_v.at[pl.ds(f * CH, CH)],
                            agg_fs[f].at[idx_r2[p]], add=True)

    _pre_copies(0, 0, False)

    def pair(j0, carry):
        k = 2 * j0
        chunk_body(k, 0, True)
        chunk_body(k + 1, 1, j0 < NFULL // 2 - 1)
        return carry

    lax.fori_loop(0, NFULL // 2, pair, None)

    off_t = base + NFULL * CH
    pltpu.sync_copy(send_hbm.at[pl.ds(off_t, TAIL)], idx_s_t)
    pltpu.sync_copy(recv_hbm.at[pl.ds(off_t, TAIL)], idx_r_t)
    for f in range(DE):
        pltpu.sync_copy(eft_hbm.at[pl.ds(f * N_EDGES + off_t, TAIL)],
                        eft2[0].at[pl.ds(f * CH, TAIL)])
    pltpu.sync_copy(g_hbm.at[idx_s_t], rows_v.at[pl.ds(0, TAIL)])
    pltpu.sync_copy(rs_sh.at[idx_s_t], rss_v.at[pl.ds(0, TAIL)])
    rsvec_t = rss_v[pl.ds(0, LANES)]
    for f in range(DE):
        wef_v[pl.ds(f * CH, LANES)] = (
            rsvec_t * eft2[0][pl.ds(f * CH, LANES)])
    pltpu.sync_copy(rows_v.at[pl.ds(0, TAIL)], acc_sh.at[idx_r_t], add=True)
    pltpu.sync_copy(rss_v.at[pl.ds(0, TAIL)], s_sh.at[idx_r_t], add=True)
    for f in range(DE):
        pltpu.sync_copy(wef_v.at[pl.ds(f * CH, TAIL)], agg_fs[f].at[idx_r_t],
                        add=True)

    plsc.subcore_barrier()
    for start, cnt in spans:
        pltpu.sync_copy(acc_sh.at[pl.ds(start, cnt)], rows_v.at[pl.ds(0, cnt)])
        pltpu.sync_copy(rows_v.at[pl.ds(0, cnt)],
                        accp_hbm.at[c, pl.ds(start, cnt)])
    pltpu.sync_copy(s_sh.at[pl.ds(r0, ZCH)], sb_v)
    pltpu.sync_copy(sb_v, sp_hbm.at[pl.ds(c * N_NODES + r0, ZCH)])
    for f in range(DE):
        pltpu.sync_copy(agg_fs[f].at[pl.ds(r0, ZCH)], sb_v)
        pltpu.sync_copy(
            sb_v, aggp_hbm.at[pl.ds((c * DE + f) * N_NODES + r0, ZCH)])

    @pl.when(t == NS - 1)
    def _():
        pltpu.sync_copy(acc_sh.at[pl.ds(NS * ZCH, ROW_TAIL)],
                        rows_v.at[pl.ds(0, ROW_TAIL)])
        pltpu.sync_copy(rows_v.at[pl.ds(0, ROW_TAIL)],
                        accp_hbm.at[c, pl.ds(NS * ZCH, ROW_TAIL)])
        pltpu.sync_copy(s_sh.at[pl.ds(NS * ZCH, ROW_TAIL)],
                        sb_v.at[pl.ds(0, ROW_TAIL)])
        pltpu.sync_copy(sb_v.at[pl.ds(0, ROW_TAIL)],
                        sp_hbm.at[pl.ds(c * N_NODES + NS * ZCH, ROW_TAIL)])
        for f in range(DE):
            pltpu.sync_copy(agg_fs[f].at[pl.ds(NS * ZCH, ROW_TAIL)],
                            sb_v.at[pl.ds(0, ROW_TAIL)])
            pltpu.sync_copy(
                sb_v.at[pl.ds(0, ROW_TAIL)],
                aggp_hbm.at[pl.ds((c * DE + f) * N_NODES + NS * ZCH,
                                  ROW_TAIL)])


_edge_call = pl.kernel(
    _edge_body,
    out_type=(
        jax.ShapeDtypeStruct((NC, N_NODES, D), jnp.float32),
        jax.ShapeDtypeStruct((NC * DE * N_NODES,), jnp.float32),
        jax.ShapeDtypeStruct((NC * N_NODES,), jnp.float32),
    ),
    mesh=_mesh,
    scratch_types=[
        pltpu.VMEM_SHARED((N_NODES, D), jnp.float32),
        pltpu.VMEM_SHARED((N_NODES,), jnp.float32),
        pltpu.VMEM_SHARED((N_NODES,), jnp.float32),
        tuple(pltpu.VMEM_SHARED((N_NODES,), jnp.float32) for _ in range(DE)),
        pltpu.VMEM((CH,), jnp.float32),
        tuple(pltpu.VMEM((CH,), jnp.int32) for _ in range(2)),
        tuple(pltpu.VMEM((CH,), jnp.int32) for _ in range(2)),
        pltpu.VMEM((TAIL,), jnp.int32),
        pltpu.VMEM((TAIL,), jnp.int32),
        tuple(pltpu.VMEM((DE * CH,), jnp.float32) for _ in range(2)),
        pltpu.VMEM((DE * CH,), jnp.float32),
        pltpu.VMEM((CH, D), jnp.float32),
        pltpu.VMEM((ZCH,), jnp.float32),
        (pltpu.SemaphoreType.DMA, pltpu.SemaphoreType.DMA),
    ],
)


def _final_body(accp_ref, aggx_ref, st_ref, rs_ref, we_ref, be_ref, out_ref):
    acc = accp_ref[0] + accp_ref[1]
    aggf = aggx_ref[:, 0, :] + aggx_ref[:, 1, :]
    s = st_ref[:, 0:1] + st_ref[:, 1:2]
    proj = jnp.dot(aggf, we_ref[...], preferred_element_type=jnp.float32)
    out_ref[...] = rs_ref[...] * (acc + proj + s * be_ref[...])


_final_call = pl.pallas_call(
    _final_body,
    grid=(N_NODES // BLK,),
    in_specs=[
        pl.BlockSpec((NC, BLK, D), lambda i: (0, i, 0)),
        pl.BlockSpec((BLK, NC, DE), lambda i: (i, 0, 0)),
        pl.BlockSpec((BLK, NC), lambda i: (i, 0)),
        pl.BlockSpec((BLK, 1), lambda i: (i, 0)),
        pl.BlockSpec((DE, D), lambda i: (0, 0)),
        pl.BlockSpec((1, D), lambda i: (0, 0)),
    ],
    out_specs=pl.BlockSpec((BLK, D), lambda i: (i, 0)),
    out_shape=jax.ShapeDtypeStruct((N_NODES, D), jnp.float32),
)


def kernel(node_features, senders, receivers, edge_features,
           W_kernel, W_bias, We_kernel, We_bias):
    degp = _deg_call(receivers)
    degt = degp.reshape(NC, N_NODES).T
    g, rs2 = _proj_call(node_features, W_kernel,
                        W_bias.reshape(1, D), degt)
    rs1 = rs2.reshape(N_NODES)
    eft = edge_features.T.reshape(DE * N_EDGES)
    accp, aggp, sp = _edge_call(g, rs1, senders, receivers, eft)
    aggx = aggp.reshape(NC, DE, N_NODES).transpose(2, 0, 1)
    st = sp.reshape(NC, N_NODES).T
    return _final_call(accp, aggx, st, rs2, We_kernel, We_bias.reshape(1, D))
